# initial kernel scaffold (unmeasured)
import jax
import jax.numpy as jnp
from jax import lax
from jax.experimental import pallas as pl
from jax.experimental.pallas import tpu as pltpu

N_DEV = 32
M = 1024
N = 1024
CHUNK = M // N_DEV


def kernel(A, B):
    def body(a_ref, b_ref, out_ref, comm_ref, send_sems, recv_sems, credit_sem):
        my = lax.axis_index("i")
        left = jnp.mod(my - 1, N_DEV)
        right = jnp.mod(my + 1, N_DEV)

        barrier = pltpu.get_barrier_semaphore()
        for nbr in (left, right):
            pl.semaphore_signal(
                barrier, inc=1,
                device_id=(nbr,), device_id_type=pl.DeviceIdType.MESH,
            )
        pl.semaphore_wait(barrier, 2)

        out_ref[...] = jnp.dot(
            a_ref[...], b_ref[...], preferred_element_type=jnp.float32
        )

        TOTAL = 2 * (N_DEV - 1)
        for s in range(TOTAL):
            slot = s % 2
            if s < N_DEV - 1:
                send_idx = jnp.mod(my - s, N_DEV)
                recv_idx = jnp.mod(my - s - 1, N_DEV)
            else:
                t = s - (N_DEV - 1)
                send_idx = jnp.mod(my + 1 - t, N_DEV)
                recv_idx = jnp.mod(my - t, N_DEV)

            if s >= 2:
                pl.semaphore_wait(credit_sem, 1)

            rdma = pltpu.make_async_remote_copy(
                src_ref=out_ref.at[pl.ds(send_idx * CHUNK, CHUNK), :],
                dst_ref=comm_ref.at[slot],
                send_sem=send_sems.at[slot],
                recv_sem=recv_sems.at[slot],
                device_id=(right,),
                device_id_type=pl.DeviceIdType.MESH,
            )
            rdma.start()
            rdma.wait()

            row = pl.ds(recv_idx * CHUNK, CHUNK)
            if s < N_DEV - 1:
                out_ref[row, :] = out_ref[row, :] + comm_ref[slot]
            else:
                out_ref[row, :] = comm_ref[slot]

            if s < TOTAL - 2:
                pl.semaphore_signal(
                    credit_sem, inc=1,
                    device_id=(left,), device_id_type=pl.DeviceIdType.MESH,
                )

    return pl.pallas_call(
        body,
        out_shape=jax.ShapeDtypeStruct((M, N), jnp.float32),
        in_specs=[
            pl.BlockSpec(memory_space=pltpu.VMEM),
            pl.BlockSpec(memory_space=pltpu.VMEM),
        ],
        out_specs=pl.BlockSpec(memory_space=pltpu.VMEM),
        scratch_shapes=[
            pltpu.VMEM((2, CHUNK, N), jnp.float32),
            pltpu.SemaphoreType.DMA((2,)),
            pltpu.SemaphoreType.DMA((2,)),
            pltpu.SemaphoreType.REGULAR,
        ],
        compiler_params=pltpu.CompilerParams(collective_id=0),
    )(A, B)


# baseline (device time: 270940 ns/iter reference)
import jax
import jax.numpy as jnp
from jax import lax
from jax.experimental import pallas as pl
from jax.experimental.pallas import tpu as pltpu

N_DEV = 32
M = 1024
N = 1024
CHUNK = M // N_DEV

DBG_TOTAL = None
DBG_NO_CREDIT = False


def kernel(A, B):
    def body(a_ref, b_ref, out_ref, comm_ref, send_sems, recv_sems, credit_sem):
        my = lax.axis_index("i")
        left = jnp.mod(my - 1, N_DEV)
        right = jnp.mod(my + 1, N_DEV)

        barrier = pltpu.get_barrier_semaphore()
        for nbr in (left, right):
            pl.semaphore_signal(
                barrier, inc=1,
                device_id=(nbr,), device_id_type=pl.DeviceIdType.MESH,
            )
        pl.semaphore_wait(barrier, 2)

        out_ref[...] = jnp.dot(
            a_ref[...], b_ref[...], preferred_element_type=jnp.float32
        )

        TOTAL = 2 * (N_DEV - 1) if DBG_TOTAL is None else DBG_TOTAL
        for s in range(TOTAL):
            slot = s % 2
            if s < N_DEV - 1:
                send_idx = jnp.mod(my - s, N_DEV)
                recv_idx = jnp.mod(my - s - 1, N_DEV)
            else:
                t = s - (N_DEV - 1)
                send_idx = jnp.mod(my + 1 - t, N_DEV)
                recv_idx = jnp.mod(my - t, N_DEV)

            if s >= 2 and not DBG_NO_CREDIT:
                pl.semaphore_wait(credit_sem, 1)

            rdma = pltpu.make_async_remote_copy(
                src_ref=out_ref.at[pl.ds(send_idx * CHUNK, CHUNK), :],
                dst_ref=comm_ref.at[slot],
                send_sem=send_sems.at[slot],
                recv_sem=recv_sems.at[slot],
                device_id=(right,),
                device_id_type=pl.DeviceIdType.MESH,
            )
            rdma.start()
            rdma.wait()

            row = pl.ds(recv_idx * CHUNK, CHUNK)
            if s < N_DEV - 1:
                out_ref[row, :] = out_ref[row, :] + comm_ref[slot]
            else:
                out_ref[row, :] = comm_ref[slot]

            if s < TOTAL - 2 and not DBG_NO_CREDIT:
                pl.semaphore_signal(
                    credit_sem, inc=1,
                    device_id=(left,), device_id_type=pl.DeviceIdType.MESH,
                )

    return pl.pallas_call(
        body,
        out_shape=jax.ShapeDtypeStruct((M, N), jnp.float32),
        in_specs=[
            pl.BlockSpec(memory_space=pltpu.VMEM),
            pl.BlockSpec(memory_space=pltpu.VMEM),
        ],
        out_specs=pl.BlockSpec(memory_space=pltpu.VMEM),
        scratch_shapes=[
            pltpu.VMEM((2, CHUNK, N), jnp.float32),
            pltpu.SemaphoreType.DMA((2,)),
            pltpu.SemaphoreType.DMA((2,)),
            pltpu.SemaphoreType.REGULAR,
        ],
        compiler_params=pltpu.CompilerParams(collective_id=0),
    )(A, B)


# device time: 94205 ns/iter; 2.8761x vs baseline; 2.8761x over previous
import jax
import jax.numpy as jnp
from jax import lax
from jax.experimental import pallas as pl
from jax.experimental.pallas import tpu as pltpu

N_DEV = 32
M = 1024
N = 1024
P_RING = 8
Z_RING = 4
PCH = M // P_RING
ZCH = PCH // Z_RING
HALF = N // 2


def kernel(A, B):
    def body(a_ref, b_ref, out_ref,
             comm_p_r, comm_p_l, comm_z_r, comm_z_l,
             send_p_r, recv_p_r, send_p_l, recv_p_l,
             send_z_r, recv_z_r, send_z_l, recv_z_l,
             cred_p_r, cred_p_l, cred_z_r, cred_z_l):
        my = lax.axis_index("i")
        z = my // 8
        p = my % 8
        y = p // 2
        pm = p % 4
        x = jnp.where((pm == 1) | (pm == 2), 1, 0)
        u = jnp.where(x == 0, y, 7 - y)
        v = jnp.where(z == 1, 3, jnp.where(z == 2, 1, jnp.where(z == 3, 2, 0)))

        def plane_dev(uu):
            uu = jnp.mod(uu, P_RING)
            yy = jnp.where(uu < 4, uu, 7 - uu)
            pp = jnp.where(
                uu < 4,
                2 * yy + jnp.mod(yy, 2),
                2 * yy + 1 - jnp.mod(yy, 2),
            )
            return z * 8 + pp

        def z_dev(vv):
            vv = jnp.mod(vv, Z_RING)
            zz = jnp.where(vv == 1, 2, jnp.where(vv == 2, 3, jnp.where(vv == 3, 1, 0)))
            return zz * 8 + p

        pnx = plane_dev(u + 1)
        ppv = plane_dev(u - 1)
        znx = z_dev(v + 1)
        zpv = z_dev(v - 1)

        barrier = pltpu.get_barrier_semaphore()
        for nbr in (pnx, ppv, znx, zpv):
            pl.semaphore_signal(
                barrier, inc=1,
                device_id=(nbr,), device_id_type=pl.DeviceIdType.MESH,
            )
        pl.semaphore_wait(barrier, 4)

        out_ref[...] = jnp.dot(
            a_ref[...], b_ref[...], preferred_element_type=jnp.float32
        )

        streams = [
            dict(q=u, w=v, pn=pnx, pp=ppv, zn=znx, zp=zpv, c0=0,
                 cp=comm_p_r, cz=comm_z_r, sp=send_p_r, rp=recv_p_r,
                 sz=send_z_r, rz=recv_z_r, kp=cred_p_r, kz=cred_z_r),
            dict(q=jnp.mod(-u, P_RING), w=jnp.mod(-v, Z_RING),
                 pn=ppv, pp=pnx, zn=zpv, zp=znx, c0=HALF,
                 cp=comm_p_l, cz=comm_z_l, sp=send_p_l, rp=recv_p_l,
                 sz=send_z_l, rz=recv_z_l, kp=cred_p_l, kz=cred_z_l),
        ]

        P_TOTAL = 2 * (P_RING - 1)
        Z_TOTAL = 2 * (Z_RING - 1)

        def plane_step(sp_idx):
            slot = sp_idx % 2
            reduce_phase = sp_idx < P_RING - 1
            started = []
            for st in streams:
                if reduce_phase:
                    send_k = jnp.mod(st["q"] - sp_idx, P_RING)
                    recv_k = jnp.mod(st["q"] - sp_idx - 1, P_RING)
                else:
                    t = sp_idx - (P_RING - 1)
                    send_k = jnp.mod(st["q"] + 1 - t, P_RING)
                    recv_k = jnp.mod(st["q"] - t, P_RING)
                if sp_idx >= 2:
                    pl.semaphore_wait(st["kp"], 1)
                rdma = pltpu.make_async_remote_copy(
                    src_ref=out_ref.at[pl.ds(send_k * PCH, PCH),
                                       pl.ds(st["c0"], HALF)],
                    dst_ref=st["cp"].at[slot],
                    send_sem=st["sp"].at[slot],
                    recv_sem=st["rp"].at[slot],
                    device_id=(st["pn"],),
                    device_id_type=pl.DeviceIdType.MESH,
                )
                rdma.start()
                started.append((st, rdma, recv_k))
            for st, rdma, recv_k in started:
                rdma.wait()
                rows = pl.ds(recv_k * PCH, PCH)
                cols = pl.ds(st["c0"], HALF)
                if reduce_phase:
                    out_ref[rows, cols] = out_ref[rows, cols] + st["cp"][slot]
                else:
                    out_ref[rows, cols] = st["cp"][slot]
                if sp_idx < P_TOTAL - 2:
                    pl.semaphore_signal(
                        st["kp"], inc=1,
                        device_id=(st["pp"],), device_id_type=pl.DeviceIdType.MESH,
                    )

        def z_step(sz_idx):
            slot = sz_idx % 2
            reduce_phase = sz_idx < Z_RING - 1
            started = []
            for st in streams:
                base = jnp.mod(st["q"] + 1, P_RING) * PCH
                if reduce_phase:
                    send_c = jnp.mod(st["w"] - sz_idx, Z_RING)
                    recv_c = jnp.mod(st["w"] - sz_idx - 1, Z_RING)
                else:
                    t = sz_idx - (Z_RING - 1)
                    send_c = jnp.mod(st["w"] + 1 - t, Z_RING)
                    recv_c = jnp.mod(st["w"] - t, Z_RING)
                if sz_idx >= 2:
                    pl.semaphore_wait(st["kz"], 1)
                rdma = pltpu.make_async_remote_copy(
                    src_ref=out_ref.at[pl.ds(base + send_c * ZCH, ZCH),
                                       pl.ds(st["c0"], HALF)],
                    dst_ref=st["cz"].at[slot],
                    send_sem=st["sz"].at[slot],
                    recv_sem=st["rz"].at[slot],
                    device_id=(st["zn"],),
                    device_id_type=pl.DeviceIdType.MESH,
                )
                rdma.start()
                started.append((st, rdma, base, recv_c))
            for st, rdma, base, recv_c in started:
                rdma.wait()
                rows = pl.ds(base + recv_c * ZCH, ZCH)
                cols = pl.ds(st["c0"], HALF)
                if reduce_phase:
                    out_ref[rows, cols] = out_ref[rows, cols] + st["cz"][slot]
                else:
                    out_ref[rows, cols] = st["cz"][slot]
                if sz_idx < Z_TOTAL - 2:
                    pl.semaphore_signal(
                        st["kz"], inc=1,
                        device_id=(st["zp"],), device_id_type=pl.DeviceIdType.MESH,
                    )

        for s in range(P_RING - 1):
            plane_step(s)
        for s in range(Z_TOTAL):
            z_step(s)
        for s in range(P_RING - 1, P_TOTAL):
            plane_step(s)

    return pl.pallas_call(
        body,
        out_shape=jax.ShapeDtypeStruct((M, N), jnp.float32),
        in_specs=[
            pl.BlockSpec(memory_space=pltpu.VMEM),
            pl.BlockSpec(memory_space=pltpu.VMEM),
        ],
        out_specs=pl.BlockSpec(memory_space=pltpu.VMEM),
        scratch_shapes=[
            pltpu.VMEM((2, PCH, HALF), jnp.float32),
            pltpu.VMEM((2, PCH, HALF), jnp.float32),
            pltpu.VMEM((2, ZCH, HALF), jnp.float32),
            pltpu.VMEM((2, ZCH, HALF), jnp.float32),
            pltpu.SemaphoreType.DMA((2,)),
            pltpu.SemaphoreType.DMA((2,)),
            pltpu.SemaphoreType.DMA((2,)),
            pltpu.SemaphoreType.DMA((2,)),
            pltpu.SemaphoreType.DMA((2,)),
            pltpu.SemaphoreType.DMA((2,)),
            pltpu.SemaphoreType.DMA((2,)),
            pltpu.SemaphoreType.DMA((2,)),
            pltpu.SemaphoreType.REGULAR,
            pltpu.SemaphoreType.REGULAR,
            pltpu.SemaphoreType.REGULAR,
            pltpu.SemaphoreType.REGULAR,
        ],
        compiler_params=pltpu.CompilerParams(collective_id=0),
    )(A, B)
